# Initial kernel scaffold; baseline (speedup 1.0000x reference)
import jax
import jax.numpy as jnp
from jax.experimental import pallas as pl


def kernel(x, edge_index, W1, b1, W2, b2):
    raise NotImplementedError("write your pallas kernel here")


# speedup vs baseline: 5.6373x; 5.6373x over previous
import functools

import jax
import jax.numpy as jnp
from jax import lax
from jax.experimental import pallas as pl
from jax.experimental.pallas import tpu as pltpu
from jax.experimental.pallas import tpu_sc as plsc

_LANES = 128


def _matmul(x, w):
    n, d = x.shape
    h = w.shape[1]
    bn = 1000
    def body(x_ref, w_ref, o_ref):
        o_ref[...] = jnp.dot(x_ref[...], w_ref[...],
                             preferred_element_type=jnp.float32)
    return pl.pallas_call(
        body,
        grid=(n // bn,),
        in_specs=[pl.BlockSpec((bn, d), lambda i: (i, 0)),
                  pl.BlockSpec((d, h), lambda i: (0, 0))],
        out_specs=pl.BlockSpec((bn, h), lambda i: (i, 0)),
        out_shape=jax.ShapeDtypeStruct((n, h), jnp.float32),
    )(x, w)


def _mid(p, s1, b1p, w2p):
    n, h = p.shape
    c = w2p.shape[1]
    bn = 1000
    def body(p_ref, s_ref, b_ref, w_ref, q_ref):
        hh = p_ref[...] + s_ref[0] + s_ref[1] + b_ref[...]
        hh = jnp.maximum(hh, 0.0)
        q_ref[...] = jnp.dot(hh, w_ref[...],
                             preferred_element_type=jnp.float32)
    return pl.pallas_call(
        body,
        grid=(n // bn,),
        in_specs=[pl.BlockSpec((bn, h), lambda i: (i, 0)),
                  pl.BlockSpec((2, bn, h), lambda i: (0, i, 0)),
                  pl.BlockSpec((1, h), lambda i: (0, 0)),
                  pl.BlockSpec((h, c), lambda i: (0, 0))],
        out_specs=pl.BlockSpec((bn, c), lambda i: (i, 0)),
        out_shape=jax.ShapeDtypeStruct((n, c), jnp.float32),
    )(p, s1, b1p.reshape(1, h), w2p)


def _final(q, s2, b2, n_out):
    n, h = q.shape
    bn = 1000
    def body(q_ref, s_ref, b_ref, o_ref):
        o_ref[...] = (q_ref[...] + s_ref[0] + s_ref[1])[:, :n_out] + b_ref[...]
    return pl.pallas_call(
        body,
        grid=(n // bn,),
        in_specs=[pl.BlockSpec((bn, h), lambda i: (i, 0)),
                  pl.BlockSpec((2, bn, h), lambda i: (0, i, 0)),
                  pl.BlockSpec((1, n_out), lambda i: (0, 0))],
        out_specs=pl.BlockSpec((bn, n_out), lambda i: (i, 0)),
        out_shape=jax.ShapeDtypeStruct((n, n_out), jnp.float32),
    )(q, s2, b2.reshape(1, n_out))


def _sc_segment_sum(table, src, dst, zeros_hbm):
    n, h = table.shape
    e = src.shape[0]
    info = plsc.get_sparse_core_info()
    nc, ns = info.num_cores, info.num_subcores
    nw = nc * ns
    ch = 128
    nchunk = e // ch
    kmax = -(-nchunk // nw)
    blk = 200
    nblk = n // blk
    kblk = -(-nblk // ns)

    mesh = plsc.VectorSubcoreMesh(core_axis_name="c", subcore_axis_name="s")

    @functools.partial(
        pl.kernel,
        out_type=jax.ShapeDtypeStruct((nc, n, h), jnp.float32),
        mesh=mesh,
        scratch_types=[
            pltpu.VMEM((ch,), jnp.int32),
            pltpu.VMEM((1, ch), jnp.int32),
            pltpu.VMEM((ch, h), jnp.float32),
            pltpu.VMEM_SHARED((n, h), jnp.float32),
            pltpu.SemaphoreType.DMA,
        ],
    )
    def k(table_hbm, src_hbm, dst_hbm, zero_hbm, out_hbm,
          src_v, dst_v, rows_v, acc, sem):
        c = lax.axis_index("c")
        s = lax.axis_index("s")
        w = c * ns + s

        def zbody(jj, carry):
            b = s + jj * ns
            @pl.when(b < nblk)
            def _():
                r0 = b * blk
                pltpu.sync_copy(zero_hbm.at[pl.ds(r0, blk)],
                                acc.at[pl.ds(r0, blk)])
            return carry

        lax.fori_loop(0, kblk, zbody, 0)
        plsc.subcore_barrier()

        def body(kk, carry):
            chunk = w + kk * nw
            @pl.when(chunk < nchunk)
            def _():
                e0 = chunk * ch
                pltpu.sync_copy(src_hbm.at[pl.ds(e0, ch)], src_v)
                pltpu.sync_copy(dst_hbm.at[pl.ds(e0, ch)], dst_v.at[0])
                pltpu.async_copy(table_hbm.at[src_v], rows_v, sem).wait()
                pltpu.sync_copy(rows_v, acc.at[dst_v.at[0]], add=True)
            return carry

        lax.fori_loop(0, kmax, body, 0)
        plsc.subcore_barrier()

        def obody(jj, carry):
            b = s + jj * ns
            @pl.when(b < nblk)
            def _():
                r0 = b * blk
                pltpu.sync_copy(acc.at[pl.ds(r0, blk)],
                                out_hbm.at[c, pl.ds(r0, blk)])
            return carry

        lax.fori_loop(0, kblk, obody, 0)

    return k(table, src, dst, zeros_hbm)


def kernel(x, edge_index, W1, b1, W2, b2):
    src = edge_index[0]
    dst = edge_index[1]
    n = x.shape[0]
    h = W1.shape[1]
    c_out = W2.shape[1]

    pad_h = _LANES - h
    w1p = jnp.pad(W1, ((0, 0), (0, pad_h)))
    b1p = jnp.pad(b1, (0, pad_h))
    w2p = jnp.pad(W2, ((0, pad_h), (0, _LANES - c_out)))
    zeros_hbm = jnp.zeros((n, _LANES), jnp.float32)

    p = _matmul(x, w1p)
    s1 = _sc_segment_sum(p, src, dst, zeros_hbm)
    q = _mid(p, s1, b1p, w2p)
    s2 = _sc_segment_sum(q, src, dst, zeros_hbm)
    return _final(q, s2, b2, c_out)


# speedup vs baseline: 8.0258x; 1.4237x over previous
import functools

import jax
import jax.numpy as jnp
from jax import lax
from jax.experimental import pallas as pl
from jax.experimental.pallas import tpu as pltpu
from jax.experimental.pallas import tpu_sc as plsc

_LANES = 128


def _matmul(x, w):
    n, d = x.shape
    h = w.shape[1]
    bn = 1000
    def body(x_ref, w_ref, o_ref):
        o_ref[...] = jnp.dot(x_ref[...], w_ref[...],
                             preferred_element_type=jnp.float32)
    return pl.pallas_call(
        body,
        grid=(n // bn,),
        in_specs=[pl.BlockSpec((bn, d), lambda i: (i, 0)),
                  pl.BlockSpec((d, h), lambda i: (0, 0))],
        out_specs=pl.BlockSpec((bn, h), lambda i: (i, 0)),
        out_shape=jax.ShapeDtypeStruct((n, h), jnp.float32),
    )(x, w)


def _mid(p, s1, b1p, w2p):
    n, h = p.shape
    c = w2p.shape[1]
    bn = 1000
    def body(p_ref, s_ref, b_ref, w_ref, q_ref):
        hh = p_ref[...] + s_ref[0] + s_ref[1] + b_ref[...]
        hh = jnp.maximum(hh, 0.0)
        q_ref[...] = jnp.dot(hh, w_ref[...],
                             preferred_element_type=jnp.float32)
    return pl.pallas_call(
        body,
        grid=(n // bn,),
        in_specs=[pl.BlockSpec((bn, h), lambda i: (i, 0)),
                  pl.BlockSpec((2, bn, h), lambda i: (0, i, 0)),
                  pl.BlockSpec((1, h), lambda i: (0, 0)),
                  pl.BlockSpec((h, c), lambda i: (0, 0))],
        out_specs=pl.BlockSpec((bn, c), lambda i: (i, 0)),
        out_shape=jax.ShapeDtypeStruct((n, c), jnp.float32),
    )(p, s1, b1p.reshape(1, h), w2p)


def _final(q, s2, b2, n_out):
    n, h = q.shape
    bn = 1000
    def body(q_ref, s_ref, b_ref, o_ref):
        o_ref[...] = (q_ref[...] + s_ref[0] + s_ref[1])[:, :n_out] + b_ref[...]
    return pl.pallas_call(
        body,
        grid=(n // bn,),
        in_specs=[pl.BlockSpec((bn, h), lambda i: (i, 0)),
                  pl.BlockSpec((2, bn, h), lambda i: (0, i, 0)),
                  pl.BlockSpec((1, n_out), lambda i: (0, 0))],
        out_specs=pl.BlockSpec((bn, n_out), lambda i: (i, 0)),
        out_shape=jax.ShapeDtypeStruct((n, n_out), jnp.float32),
    )(q, s2, b2.reshape(1, n_out))


def _sc_segment_sum(table, src, dst, zeros_hbm):
    n, h = table.shape
    e = src.shape[0]
    info = plsc.get_sparse_core_info()
    nc, ns = info.num_cores, info.num_subcores
    nw = nc * ns
    ch = 128
    nchunk = e // ch
    kmax = -(-nchunk // nw)
    blk = 200
    nblk = n // blk
    kblk = -(-nblk // ns)

    mesh = plsc.VectorSubcoreMesh(core_axis_name="c", subcore_axis_name="s")

    @functools.partial(
        pl.kernel,
        out_type=jax.ShapeDtypeStruct((nc, n, h), jnp.float32),
        mesh=mesh,
        scratch_types=[
            pltpu.VMEM((2, ch), jnp.int32),
            pltpu.VMEM((2, ch), jnp.int32),
            pltpu.VMEM((2, ch, h), jnp.float32),
            pltpu.VMEM_SHARED((n, h), jnp.float32),
            pltpu.SemaphoreType.DMA,
        ],
    )
    def k(table_hbm, src_hbm, dst_hbm, zero_hbm, out_hbm,
          si, di, rows, acc, sem):
        c = lax.axis_index("c")
        s = lax.axis_index("s")
        w = c * ns + s

        def zbody(jj, carry):
            b = s + jj * ns
            @pl.when(b < nblk)
            def _():
                r0 = b * blk
                pltpu.sync_copy(zero_hbm.at[pl.ds(r0, blk)],
                                acc.at[pl.ds(r0, blk)])
            return carry

        lax.fori_loop(0, kblk, zbody, 0)
        plsc.subcore_barrier()

        e0 = w * ch
        pltpu.sync_copy(src_hbm.at[pl.ds(e0, ch)], si.at[0])
        pltpu.sync_copy(dst_hbm.at[pl.ds(e0, ch)], di.at[0])
        pltpu.async_copy(table_hbm.at[si.at[0]], rows.at[0], sem)

        def body(kk, carry):
            m = lax.rem(kk, 2)
            mn = lax.rem(kk + 1, 2)
            nxt = w + (kk + 1) * nw
            @pl.when(nxt < nchunk)
            def _():
                e1 = nxt * ch
                pltpu.sync_copy(src_hbm.at[pl.ds(e1, ch)], si.at[mn])
                pltpu.sync_copy(dst_hbm.at[pl.ds(e1, ch)], di.at[mn])
                pltpu.async_copy(table_hbm.at[si.at[mn]], rows.at[mn], sem)
            @pl.when(w + kk * nw < nchunk)
            def _():
                pltpu.make_async_copy(table_hbm.at[si.at[m]], rows.at[m],
                                      sem).wait()
                pltpu.sync_copy(rows.at[m], acc.at[di.at[m]], add=True)
            return carry

        lax.fori_loop(0, kmax, body, 0)
        plsc.subcore_barrier()

        def obody(jj, carry):
            b = s + jj * ns
            @pl.when(b < nblk)
            def _():
                r0 = b * blk
                pltpu.sync_copy(acc.at[pl.ds(r0, blk)],
                                out_hbm.at[c, pl.ds(r0, blk)])
            return carry

        lax.fori_loop(0, kblk, obody, 0)

    return k(table, src, dst, zeros_hbm)


def kernel(x, edge_index, W1, b1, W2, b2):
    src = edge_index[0]
    dst = edge_index[1]
    n = x.shape[0]
    h = W1.shape[1]
    c_out = W2.shape[1]

    pad_h = _LANES - h
    w1p = jnp.pad(W1, ((0, 0), (0, pad_h)))
    b1p = jnp.pad(b1, (0, pad_h))
    w2p = jnp.pad(W2, ((0, pad_h), (0, _LANES - c_out)))
    zeros_hbm = jnp.zeros((n, _LANES), jnp.float32)

    p = _matmul(x, w1p)
    s1 = _sc_segment_sum(p, src, dst, zeros_hbm)
    q = _mid(p, s1, b1p, w2p)
    s2 = _sc_segment_sum(q, src, dst, zeros_hbm)
    return _final(q, s2, b2, c_out)


# speedup vs baseline: 10.1103x; 1.2597x over previous
import functools

import jax
import jax.numpy as jnp
from jax import lax
from jax.experimental import pallas as pl
from jax.experimental.pallas import tpu as pltpu
from jax.experimental.pallas import tpu_sc as plsc

_LANES = 128


def _matmul(x, w):
    n, d = x.shape
    h = w.shape[1]
    bn = 1000
    def body(x_ref, w_ref, o_ref):
        o_ref[...] = jnp.dot(x_ref[...], w_ref[...],
                             preferred_element_type=jnp.float32)
    return pl.pallas_call(
        body,
        grid=(n // bn,),
        in_specs=[pl.BlockSpec((bn, d), lambda i: (i, 0)),
                  pl.BlockSpec((d, h), lambda i: (0, 0))],
        out_specs=pl.BlockSpec((bn, h), lambda i: (i, 0)),
        out_shape=jax.ShapeDtypeStruct((n, h), jnp.float32),
    )(x, w)


def _mid(p, s1, b1p, w2p):
    n, h = p.shape
    c = w2p.shape[1]
    bn = 1000
    def body(p_ref, s_ref, b_ref, w_ref, q_ref):
        hh = p_ref[...] + s_ref[0] + s_ref[1] + b_ref[...]
        hh = jnp.maximum(hh, 0.0)
        q_ref[...] = jnp.dot(hh, w_ref[...],
                             preferred_element_type=jnp.float32)
    return pl.pallas_call(
        body,
        grid=(n // bn,),
        in_specs=[pl.BlockSpec((bn, h), lambda i: (i, 0)),
                  pl.BlockSpec((2, bn, h), lambda i: (0, i, 0)),
                  pl.BlockSpec((1, h), lambda i: (0, 0)),
                  pl.BlockSpec((h, c), lambda i: (0, 0))],
        out_specs=pl.BlockSpec((bn, c), lambda i: (i, 0)),
        out_shape=jax.ShapeDtypeStruct((n, c), jnp.float32),
    )(p, s1, b1p.reshape(1, h), w2p)


def _final(q, s2, b2, n_out):
    n, h = q.shape
    bn = 1000
    def body(q_ref, s_ref, b_ref, o_ref):
        o_ref[...] = (q_ref[...] + s_ref[0] + s_ref[1])[:, :n_out] + b_ref[...]
    return pl.pallas_call(
        body,
        grid=(n // bn,),
        in_specs=[pl.BlockSpec((bn, h), lambda i: (i, 0)),
                  pl.BlockSpec((2, bn, h), lambda i: (0, i, 0)),
                  pl.BlockSpec((1, n_out), lambda i: (0, 0))],
        out_specs=pl.BlockSpec((bn, n_out), lambda i: (i, 0)),
        out_shape=jax.ShapeDtypeStruct((n, n_out), jnp.float32),
    )(q, s2, b2.reshape(1, n_out))


def _sc_segment_sum(table, src, dst, zeros_hbm):
    n, h = table.shape
    e = src.shape[0]
    info = plsc.get_sparse_core_info()
    nc, ns = info.num_cores, info.num_subcores
    nw = nc * ns
    ch = 80
    nchunk = e // ch
    kmax = -(-nchunk // nw)
    blk = 200
    nblk = n // blk
    kblk = -(-nblk // ns)

    nb = 4
    mesh = plsc.VectorSubcoreMesh(core_axis_name="c", subcore_axis_name="s")

    @functools.partial(
        pl.kernel,
        out_type=jax.ShapeDtypeStruct((nc, n, h), jnp.float32),
        mesh=mesh,
        scratch_types=[
            pltpu.VMEM((nb, ch), jnp.int32),
            pltpu.VMEM((nb, ch), jnp.int32),
            pltpu.VMEM((nb, ch, h), jnp.float32),
            pltpu.VMEM_SHARED((n, h), jnp.float32),
            pltpu.SemaphoreType.DMA((nb,)),
            pltpu.SemaphoreType.DMA((nb,)),
            pltpu.SemaphoreType.DMA((nb,)),
        ],
    )
    def k(table_hbm, src_hbm, dst_hbm, zero_hbm, out_hbm,
          si, di, rows, acc, sem_i, sem_g, sem_s):
        c = lax.axis_index("c")
        s = lax.axis_index("s")
        w = c * ns + s

        def zbody(jj, carry):
            b = s + jj * ns
            @pl.when(b < nblk)
            def _():
                r0 = b * blk
                pltpu.sync_copy(zero_hbm.at[pl.ds(r0, blk)],
                                acc.at[pl.ds(r0, blk)])
            return carry

        lax.fori_loop(0, kblk, zbody, 0)
        plsc.subcore_barrier()

        def fire_idx(kk):
            m = lax.rem(kk, nb)
            e0 = (w + kk * nw) * ch
            pltpu.async_copy(src_hbm.at[pl.ds(e0, ch)], si.at[m], sem_i.at[m])
            pltpu.async_copy(dst_hbm.at[pl.ds(e0, ch)], di.at[m], sem_i.at[m])

        def valid(kk):
            return (kk >= 0) & (w + kk * nw < nchunk)

        fire_idx(0)
        fire_idx(1)

        def body(kk, carry):
            m = lax.rem(kk, nb)
            m1 = lax.rem(kk + 1, nb)
            m2 = lax.rem(kk + 2, nb)
            @pl.when(valid(kk - 2))
            def _():
                pltpu.make_async_copy(rows.at[m2], acc.at[di.at[m2]],
                                      sem_s.at[m2]).wait()
            @pl.when(valid(kk + 2))
            def _():
                fire_idx(kk + 2)
            @pl.when(valid(kk + 1))
            def _():
                pltpu.make_async_copy(src_hbm.at[pl.ds(0, ch)], si.at[m1],
                                      sem_i.at[m1]).wait()
                pltpu.make_async_copy(dst_hbm.at[pl.ds(0, ch)], di.at[m1],
                                      sem_i.at[m1]).wait()
                pltpu.async_copy(table_hbm.at[si.at[m1]], rows.at[m1],
                                 sem_g.at[m1])
            @pl.when(valid(kk))
            def _():
                pltpu.make_async_copy(table_hbm.at[si.at[m]], rows.at[m],
                                      sem_g.at[m]).wait()
                pltpu.async_copy(rows.at[m], acc.at[di.at[m]], sem_s.at[m],
                                 add=True)
            return carry

        m0 = 0
        pltpu.make_async_copy(src_hbm.at[pl.ds(0, ch)], si.at[m0],
                              sem_i.at[m0]).wait()
        pltpu.make_async_copy(dst_hbm.at[pl.ds(0, ch)], di.at[m0],
                              sem_i.at[m0]).wait()
        pltpu.async_copy(table_hbm.at[si.at[m0]], rows.at[m0], sem_g.at[m0])

        lax.fori_loop(0, kmax + 2, body, 0)
        plsc.subcore_barrier()

        def obody(jj, carry):
            b = s + jj * ns
            @pl.when(b < nblk)
            def _():
                r0 = b * blk
                pltpu.sync_copy(acc.at[pl.ds(r0, blk)],
                                out_hbm.at[c, pl.ds(r0, blk)])
            return carry

        lax.fori_loop(0, kblk, obody, 0)

    return k(table, src, dst, zeros_hbm)


def kernel(x, edge_index, W1, b1, W2, b2):
    src = edge_index[0]
    dst = edge_index[1]
    n = x.shape[0]
    h = W1.shape[1]
    c_out = W2.shape[1]

    pad_h = _LANES - h
    w1p = jnp.pad(W1, ((0, 0), (0, pad_h)))
    b1p = jnp.pad(b1, (0, pad_h))
    w2p = jnp.pad(W2, ((0, pad_h), (0, _LANES - c_out)))
    zeros_hbm = jnp.zeros((n, _LANES), jnp.float32)

    p = _matmul(x, w1p)
    s1 = _sc_segment_sum(p, src, dst, zeros_hbm)
    q = _mid(p, s1, b1p, w2p)
    s2 = _sc_segment_sum(q, src, dst, zeros_hbm)
    return _final(q, s2, b2, c_out)


# speedup vs baseline: 10.1505x; 1.0040x over previous
import functools

import jax
import jax.numpy as jnp
from jax import lax
from jax.experimental import pallas as pl
from jax.experimental.pallas import tpu as pltpu
from jax.experimental.pallas import tpu_sc as plsc

_LANES = 128


def _matmul(x, w):
    n, d = x.shape
    h = w.shape[1]
    bn = 1000
    def body(x_ref, w_ref, o_ref):
        o_ref[...] = jnp.dot(x_ref[...], w_ref[...],
                             preferred_element_type=jnp.float32)
    return pl.pallas_call(
        body,
        grid=(n // bn,),
        in_specs=[pl.BlockSpec((bn, d), lambda i: (i, 0)),
                  pl.BlockSpec((d, h), lambda i: (0, 0))],
        out_specs=pl.BlockSpec((bn, h), lambda i: (i, 0)),
        out_shape=jax.ShapeDtypeStruct((n, h), jnp.float32),
    )(x, w)


def _mid(p, s1, b1p, w2p):
    n, hp = p.shape
    cq = w2p.shape[1]
    bn = 1000
    def body(p_ref, s_ref, b_ref, w_ref, q_ref):
        hh = p_ref[...] + s_ref[0] + s_ref[1] + b_ref[...]
        hh = jnp.maximum(hh, 0.0)
        q_ref[...] = jnp.dot(hh, w_ref[...],
                             preferred_element_type=jnp.float32)
    return pl.pallas_call(
        body,
        grid=(n // bn,),
        in_specs=[pl.BlockSpec((bn, hp), lambda i: (i, 0)),
                  pl.BlockSpec((2, bn, hp), lambda i: (0, i, 0)),
                  pl.BlockSpec((1, hp), lambda i: (0, 0)),
                  pl.BlockSpec((hp, cq), lambda i: (0, 0))],
        out_specs=pl.BlockSpec((bn, cq), lambda i: (i, 0)),
        out_shape=jax.ShapeDtypeStruct((n, cq), jnp.float32),
    )(p, s1, b1p.reshape(1, hp), w2p)


def _final(q, s2, b2, c_out):
    n, hp = q.shape
    bn = 1000
    def body(q_ref, s_ref, b_ref, o_ref):
        o_ref[...] = (q_ref[...] + s_ref[0] + s_ref[1])[:, :c_out] + b_ref[...]
    return pl.pallas_call(
        body,
        grid=(n // bn,),
        in_specs=[pl.BlockSpec((bn, hp), lambda i: (i, 0)),
                  pl.BlockSpec((2, bn, hp), lambda i: (0, i, 0)),
                  pl.BlockSpec((1, c_out), lambda i: (0, 0))],
        out_specs=pl.BlockSpec((bn, c_out), lambda i: (i, 0)),
        out_shape=jax.ShapeDtypeStruct((n, c_out), jnp.float32),
    )(q, s2, b2.reshape(1, c_out))


def _sc_segment_sum(table, src, dst, zeros_hbm):
    n, h = table.shape
    e = src.shape[0]
    info = plsc.get_sparse_core_info()
    nc, ns = info.num_cores, info.num_subcores
    nw = nc * ns
    ch = 80
    nchunk = e // ch
    kmax = -(-nchunk // nw)
    blk = 200
    nblk = n // blk
    kblk = -(-nblk // ns)
    nb = 4

    mesh = plsc.VectorSubcoreMesh(core_axis_name="c", subcore_axis_name="s")

    @functools.partial(
        pl.kernel,
        out_type=jax.ShapeDtypeStruct((nc, n, h), jnp.float32),
        mesh=mesh,
        scratch_types=[
            pltpu.VMEM((nb, ch), jnp.int32),
            pltpu.VMEM((nb, ch), jnp.int32),
            pltpu.VMEM((nb, ch, h), jnp.float32),
            pltpu.VMEM_SHARED((n, h), jnp.float32),
            pltpu.SemaphoreType.DMA((nb,)),
            pltpu.SemaphoreType.DMA((nb,)),
            pltpu.SemaphoreType.DMA((nb,)),
        ],
    )
    def k(table_hbm, src_hbm, dst_hbm, zero_hbm, out_hbm,
          si, di, rows, acc, sem_i, sem_g, sem_s):
        c = lax.axis_index("c")
        s = lax.axis_index("s")
        w = c * ns + s

        def zbody(jj, carry):
            b = s + jj * ns
            @pl.when(b < nblk)
            def _():
                r0 = b * blk
                pltpu.sync_copy(zero_hbm.at[pl.ds(r0, blk)],
                                acc.at[pl.ds(r0, blk)])
            return carry

        lax.fori_loop(0, kblk, zbody, 0)
        plsc.subcore_barrier()

        def fire_idx(kk):
            m = lax.rem(kk, nb)
            e0 = (w + kk * nw) * ch
            pltpu.async_copy(src_hbm.at[pl.ds(e0, ch)], si.at[m], sem_i.at[m])
            pltpu.async_copy(dst_hbm.at[pl.ds(e0, ch)], di.at[m], sem_i.at[m])

        def valid(kk):
            return (kk >= 0) & (w + kk * nw < nchunk)

        fire_idx(0)
        fire_idx(1)

        def body(kk, carry):
            m = lax.rem(kk, nb)
            m1 = lax.rem(kk + 1, nb)
            m2 = lax.rem(kk + 2, nb)
            @pl.when(valid(kk - 2))
            def _():
                pltpu.make_async_copy(rows.at[m2], acc.at[di.at[m2]],
                                      sem_s.at[m2]).wait()
            @pl.when(valid(kk + 2))
            def _():
                fire_idx(kk + 2)
            @pl.when(valid(kk + 1))
            def _():
                pltpu.make_async_copy(src_hbm.at[pl.ds(0, ch)], si.at[m1],
                                      sem_i.at[m1]).wait()
                pltpu.make_async_copy(dst_hbm.at[pl.ds(0, ch)], di.at[m1],
                                      sem_i.at[m1]).wait()
                pltpu.async_copy(table_hbm.at[si.at[m1]], rows.at[m1],
                                 sem_g.at[m1])
            @pl.when(valid(kk))
            def _():
                pltpu.make_async_copy(table_hbm.at[si.at[m]], rows.at[m],
                                      sem_g.at[m]).wait()
                pltpu.async_copy(rows.at[m], acc.at[di.at[m]], sem_s.at[m],
                                 add=True)
            return carry

        m0 = 0
        pltpu.make_async_copy(src_hbm.at[pl.ds(0, ch)], si.at[m0],
                              sem_i.at[m0]).wait()
        pltpu.make_async_copy(dst_hbm.at[pl.ds(0, ch)], di.at[m0],
                              sem_i.at[m0]).wait()
        pltpu.async_copy(table_hbm.at[si.at[m0]], rows.at[m0], sem_g.at[m0])

        lax.fori_loop(0, kmax + 2, body, 0)
        plsc.subcore_barrier()

        def obody(jj, carry):
            b = s + jj * ns
            @pl.when(b < nblk)
            def _():
                r0 = b * blk
                pltpu.sync_copy(acc.at[pl.ds(r0, blk)],
                                out_hbm.at[c, pl.ds(r0, blk)])
            return carry

        lax.fori_loop(0, kblk, obody, 0)

    return k(table, src, dst, zeros_hbm)


def kernel(x, edge_index, W1, b1, W2, b2):
    src = edge_index[0]
    dst = edge_index[1]
    n = x.shape[0]
    h = W1.shape[1]
    c_out = W2.shape[1]

    w1p = jnp.pad(W1, ((0, 0), (0, _LANES - h)))
    b1p = jnp.pad(b1, (0, _LANES - h))
    w2p = jnp.pad(W2, ((0, _LANES - h), (0, _LANES - c_out)))
    zeros_hbm = jnp.zeros((n, _LANES), jnp.float32)

    p = _matmul(x, w1p)
    s1 = _sc_segment_sum(p, src, dst, zeros_hbm)
    q = _mid(p, s1, b1p, w2p)
    s2 = _sc_segment_sum(q, src, dst, zeros_hbm)
    return _final(q, s2, b2, c_out)


# speedup vs baseline: 11.4203x; 1.1251x over previous
import functools

import jax
import jax.numpy as jnp
from jax import lax
from jax.experimental import pallas as pl
from jax.experimental.pallas import tpu as pltpu
from jax.experimental.pallas import tpu_sc as plsc

_LANES = 128


def _matmul(x, w):
    n, d = x.shape
    h = w.shape[1]
    bn = 1000
    def body(x_ref, w_ref, o_ref):
        o_ref[...] = jnp.dot(x_ref[...], w_ref[...],
                             preferred_element_type=jnp.float32)
    return pl.pallas_call(
        body,
        grid=(n // bn,),
        in_specs=[pl.BlockSpec((bn, d), lambda i: (i, 0)),
                  pl.BlockSpec((d, h), lambda i: (0, 0))],
        out_specs=pl.BlockSpec((bn, h), lambda i: (i, 0)),
        out_shape=jax.ShapeDtypeStruct((n, h), jnp.float32),
    )(x, w)


def _mid(p, s1, b1p, w2p):
    n, hp = p.shape
    cq = w2p.shape[1]
    bn = 1000
    def body(p_ref, s_ref, b_ref, w_ref, q_ref):
        hh = p_ref[...] + s_ref[0] + s_ref[1] + b_ref[...]
        hh = jnp.maximum(hh, 0.0)
        q_ref[...] = jnp.dot(hh, w_ref[...],
                             preferred_element_type=jnp.float32)
    return pl.pallas_call(
        body,
        grid=(n // bn,),
        in_specs=[pl.BlockSpec((bn, hp), lambda i: (i, 0)),
                  pl.BlockSpec((2, bn, hp), lambda i: (0, i, 0)),
                  pl.BlockSpec((1, hp), lambda i: (0, 0)),
                  pl.BlockSpec((hp, cq), lambda i: (0, 0))],
        out_specs=pl.BlockSpec((bn, cq), lambda i: (i, 0)),
        out_shape=jax.ShapeDtypeStruct((n, cq), jnp.float32),
    )(p, s1, b1p.reshape(1, hp), w2p)


def _final(q, s2, b2, c_out):
    n, hp = q.shape
    bn = 1000
    def body(q_ref, s_ref, b_ref, o_ref):
        o_ref[...] = (q_ref[...] + s_ref[0] + s_ref[1])[:, :c_out] + b_ref[...]
    return pl.pallas_call(
        body,
        grid=(n // bn,),
        in_specs=[pl.BlockSpec((bn, hp), lambda i: (i, 0)),
                  pl.BlockSpec((2, bn, hp), lambda i: (0, i, 0)),
                  pl.BlockSpec((1, c_out), lambda i: (0, 0))],
        out_specs=pl.BlockSpec((bn, c_out), lambda i: (i, 0)),
        out_shape=jax.ShapeDtypeStruct((n, c_out), jnp.float32),
    )(q, s2, b2.reshape(1, c_out))


def _sc_segment_sum(table, src, dst, zeros_hbm):
    n, h = table.shape
    e = src.shape[0]
    info = plsc.get_sparse_core_info()
    nc, ns = info.num_cores, info.num_subcores
    nw = nc * ns
    ch = 128
    nchunk = e // ch
    kmax = -(-nchunk // nw)
    blk = 200
    nblk = n // blk
    kblk = -(-nblk // ns)
    nb = 4

    mesh = plsc.VectorSubcoreMesh(core_axis_name="c", subcore_axis_name="s")

    @functools.partial(
        pl.kernel,
        out_type=jax.ShapeDtypeStruct((nc, n, h), jnp.float32),
        mesh=mesh,
        compiler_params=pltpu.CompilerParams(use_tc_tiling_on_sc=False),
        scratch_types=[
            pltpu.VMEM((nb, ch), jnp.int32),
            pltpu.VMEM((nb, ch), jnp.int32),
            pltpu.VMEM((nb, ch, h), jnp.float32),
            pltpu.VMEM_SHARED((n, h), jnp.float32),
            pltpu.SemaphoreType.DMA((nb,)),
            pltpu.SemaphoreType.DMA((nb,)),
            pltpu.SemaphoreType.DMA((nb,)),
        ],
    )
    def k(table_hbm, src_hbm, dst_hbm, zero_hbm, out_hbm,
          si, di, rows, acc, sem_i, sem_g, sem_s):
        c = lax.axis_index("c")
        s = lax.axis_index("s")
        w = c * ns + s

        def zbody(jj, carry):
            b = s + jj * ns
            @pl.when(b < nblk)
            def _():
                r0 = b * blk
                pltpu.sync_copy(zero_hbm.at[pl.ds(r0, blk)],
                                acc.at[pl.ds(r0, blk)])
            return carry

        lax.fori_loop(0, kblk, zbody, 0)
        plsc.subcore_barrier()

        def fire_idx(kk):
            m = lax.rem(kk, nb)
            e0 = (w + kk * nw) * ch
            pltpu.async_copy(src_hbm.at[pl.ds(e0, ch)], si.at[m], sem_i.at[m])
            pltpu.async_copy(dst_hbm.at[pl.ds(e0, ch)], di.at[m], sem_i.at[m])

        def valid(kk):
            return (kk >= 0) & (w + kk * nw < nchunk)

        fire_idx(0)
        fire_idx(1)

        def body(kk, carry):
            m = lax.rem(kk, nb)
            m1 = lax.rem(kk + 1, nb)
            m2 = lax.rem(kk + 2, nb)
            @pl.when(valid(kk - 2))
            def _():
                pltpu.make_async_copy(rows.at[m2], acc.at[di.at[m2]],
                                      sem_s.at[m2]).wait()
            @pl.when(valid(kk + 2))
            def _():
                fire_idx(kk + 2)
            @pl.when(valid(kk + 1))
            def _():
                pltpu.make_async_copy(src_hbm.at[pl.ds(0, ch)], si.at[m1],
                                      sem_i.at[m1]).wait()
                pltpu.make_async_copy(dst_hbm.at[pl.ds(0, ch)], di.at[m1],
                                      sem_i.at[m1]).wait()
                pltpu.async_copy(table_hbm.at[si.at[m1]], rows.at[m1],
                                 sem_g.at[m1])
            @pl.when(valid(kk))
            def _():
                pltpu.make_async_copy(table_hbm.at[si.at[m]], rows.at[m],
                                      sem_g.at[m]).wait()
                pltpu.async_copy(rows.at[m], acc.at[di.at[m]], sem_s.at[m],
                                 add=True)
            return carry

        m0 = 0
        pltpu.make_async_copy(src_hbm.at[pl.ds(0, ch)], si.at[m0],
                              sem_i.at[m0]).wait()
        pltpu.make_async_copy(dst_hbm.at[pl.ds(0, ch)], di.at[m0],
                              sem_i.at[m0]).wait()
        pltpu.async_copy(table_hbm.at[si.at[m0]], rows.at[m0], sem_g.at[m0])

        lax.fori_loop(0, kmax + 2, body, 0)
        plsc.subcore_barrier()

        def obody(jj, carry):
            b = s + jj * ns
            @pl.when(b < nblk)
            def _():
                r0 = b * blk
                pltpu.sync_copy(acc.at[pl.ds(r0, blk)],
                                out_hbm.at[c, pl.ds(r0, blk)])
            return carry

        lax.fori_loop(0, kblk, obody, 0)

    return k(table, src, dst, zeros_hbm)


def kernel(x, edge_index, W1, b1, W2, b2):
    src = edge_index[0]
    dst = edge_index[1]
    n = x.shape[0]
    h = W1.shape[1]
    c_out = W2.shape[1]

    zeros_hbm = jnp.zeros((n, h), jnp.float32)

    p = _matmul(x, W1)
    s1 = _sc_segment_sum(p, src, dst, zeros_hbm)
    q = _mid(p, s1, b1, W2)
    s2 = _sc_segment_sum(q, src, dst, zeros_hbm)
    return _final(q, s2, b2, c_out)


# speedup vs baseline: 15.0379x; 1.3168x over previous
import functools

import jax
import jax.numpy as jnp
from jax import lax
from jax.experimental import pallas as pl
from jax.experimental.pallas import tpu as pltpu
from jax.experimental.pallas import tpu_sc as plsc


def _matmul(x, w):
    n, d = x.shape
    h = w.shape[1]
    bn = 2000
    def body(x_ref, w_ref, o_ref):
        o_ref[...] = jnp.dot(x_ref[...], w_ref[...],
                             preferred_element_type=jnp.float32)
    return pl.pallas_call(
        body,
        grid=(n // bn,),
        in_specs=[pl.BlockSpec((bn, d), lambda i: (i, 0)),
                  pl.BlockSpec((d, h), lambda i: (0, 0))],
        out_specs=pl.BlockSpec((bn, h), lambda i: (i, 0)),
        out_shape=jax.ShapeDtypeStruct((n, h), jnp.float32),
    )(x, w)


def _mid_pairs(pp, s1p, b1pair, w2d):
    n2, h2 = pp.shape
    bn = 1000
    def body(p_ref, s_ref, b_ref, w_ref, q_ref):
        hh = p_ref[...] + s_ref[0] + s_ref[1] + b_ref[...]
        hh = jnp.maximum(hh, 0.0)
        q_ref[...] = jnp.dot(hh, w_ref[...],
                             preferred_element_type=jnp.float32)
    return pl.pallas_call(
        body,
        grid=(n2 // bn,),
        in_specs=[pl.BlockSpec((bn, h2), lambda i: (i, 0)),
                  pl.BlockSpec((2, bn, h2), lambda i: (0, i, 0)),
                  pl.BlockSpec((1, h2), lambda i: (0, 0)),
                  pl.BlockSpec((h2, h2), lambda i: (0, 0))],
        out_specs=pl.BlockSpec((bn, h2), lambda i: (i, 0)),
        out_shape=jax.ShapeDtypeStruct((n2, h2), jnp.float32),
    )(pp, s1p, b1pair, w2d)


def _final_pairs(qp, s2p, b2pair):
    n2, h2 = qp.shape
    bn = 1000
    def body(q_ref, s_ref, b_ref, o_ref):
        o_ref[...] = q_ref[...] + s_ref[0] + s_ref[1] + b_ref[...]
    return pl.pallas_call(
        body,
        grid=(n2 // bn,),
        in_specs=[pl.BlockSpec((bn, h2), lambda i: (i, 0)),
                  pl.BlockSpec((2, bn, h2), lambda i: (0, i, 0)),
                  pl.BlockSpec((1, h2), lambda i: (0, 0))],
        out_specs=pl.BlockSpec((bn, h2), lambda i: (i, 0)),
        out_shape=jax.ShapeDtypeStruct((n2, h2), jnp.float32),
    )(qp, s2p, b2pair)


def _sc_segment_sum(table, ei, zeros_hbm):
    n, h = table.shape
    e = ei.shape[1]
    info = plsc.get_sparse_core_info()
    nc, ns = info.num_cores, info.num_subcores
    nw = nc * ns
    ch = 128
    nchunk = e // ch
    kmax = -(-nchunk // nw)
    blk = 200
    nblk = n // blk
    kblk = -(-nblk // ns)
    nb = 4

    mesh = plsc.VectorSubcoreMesh(core_axis_name="c", subcore_axis_name="s")

    @functools.partial(
        pl.kernel,
        out_type=jax.ShapeDtypeStruct((nc, n, h), jnp.float32),
        mesh=mesh,
        compiler_params=pltpu.CompilerParams(use_tc_tiling_on_sc=False),
        scratch_types=[
            pltpu.VMEM((nb, ch), jnp.int32),
            pltpu.VMEM((nb, ch), jnp.int32),
            pltpu.VMEM((nb, ch, h), jnp.float32),
            pltpu.VMEM_SHARED((n, h), jnp.float32),
            pltpu.SemaphoreType.DMA((nb,)),
            pltpu.SemaphoreType.DMA((nb,)),
            pltpu.SemaphoreType.DMA((nb,)),
        ],
    )
    def k(table_hbm, ei_hbm, zero_hbm, out_hbm,
          si, di, rows, acc, sem_i, sem_g, sem_s):
        c = lax.axis_index("c")
        s = lax.axis_index("s")
        w = c * ns + s

        def zbody(jj, carry):
            b = s + jj * ns
            @pl.when(b < nblk)
            def _():
                r0 = b * blk
                pltpu.sync_copy(zero_hbm.at[pl.ds(r0, blk)],
                                acc.at[pl.ds(r0, blk)])
            return carry

        lax.fori_loop(0, kblk, zbody, 0)
        plsc.subcore_barrier()

        def fire_idx(kk):
            m = lax.rem(kk, nb)
            e0 = (w + kk * nw) * ch
            pltpu.async_copy(ei_hbm.at[0, pl.ds(e0, ch)], si.at[m],
                             sem_i.at[m])
            pltpu.async_copy(ei_hbm.at[1, pl.ds(e0, ch)], di.at[m],
                             sem_i.at[m])

        def valid(kk):
            return (kk >= 0) & (w + kk * nw < nchunk)

        fire_idx(0)
        fire_idx(1)

        def body(kk, carry):
            m = lax.rem(kk, nb)
            m1 = lax.rem(kk + 1, nb)
            m2 = lax.rem(kk + 2, nb)
            @pl.when(valid(kk - 2))
            def _():
                pltpu.make_async_copy(rows.at[m2], acc.at[di.at[m2]],
                                      sem_s.at[m2]).wait()
            @pl.when(valid(kk + 2))
            def _():
                fire_idx(kk + 2)
            @pl.when(valid(kk + 1))
            def _():
                pltpu.make_async_copy(ei_hbm.at[0, pl.ds(0, ch)], si.at[m1],
                                      sem_i.at[m1]).wait()
                pltpu.make_async_copy(ei_hbm.at[1, pl.ds(0, ch)], di.at[m1],
                                      sem_i.at[m1]).wait()
                pltpu.async_copy(table_hbm.at[si.at[m1]], rows.at[m1],
                                 sem_g.at[m1])
            @pl.when(valid(kk))
            def _():
                pltpu.make_async_copy(table_hbm.at[si.at[m]], rows.at[m],
                                      sem_g.at[m]).wait()
                pltpu.async_copy(rows.at[m], acc.at[di.at[m]], sem_s.at[m],
                                 add=True)
            return carry

        m0 = 0
        pltpu.make_async_copy(ei_hbm.at[0, pl.ds(0, ch)], si.at[m0],
                              sem_i.at[m0]).wait()
        pltpu.make_async_copy(ei_hbm.at[1, pl.ds(0, ch)], di.at[m0],
                              sem_i.at[m0]).wait()
        pltpu.async_copy(table_hbm.at[si.at[m0]], rows.at[m0], sem_g.at[m0])

        lax.fori_loop(0, kmax + 2, body, 0)
        plsc.subcore_barrier()

        def obody(jj, carry):
            b = s + jj * ns
            @pl.when(b < nblk)
            def _():
                r0 = b * blk
                pltpu.sync_copy(acc.at[pl.ds(r0, blk)],
                                out_hbm.at[c, pl.ds(r0, blk)])
            return carry

        lax.fori_loop(0, kblk, obody, 0)

    return k(table, ei, zeros_hbm)


def kernel(x, edge_index, W1, b1, W2, b2):
    n = x.shape[0]
    h = W1.shape[1]
    c_out = W2.shape[1]
    n2 = n // 2
    h2 = 2 * h

    w2d = jnp.zeros((h2, h2), W2.dtype)
    w2d = w2d.at[:h, :c_out].set(W2).at[h:, c_out:].set(W2)
    b1pair = jnp.concatenate([b1, b1]).reshape(1, h2)
    b2pair = jnp.concatenate([b2, b2]).reshape(1, h2)
    zeros_hbm = jnp.zeros((n, h), jnp.float32)

    p = _matmul(x, W1)
    s1 = _sc_segment_sum(p, edge_index, zeros_hbm)
    qp = _mid_pairs(p.reshape(n2, h2), s1.reshape(2, n2, h2), b1pair, w2d)
    s2 = _sc_segment_sum(qp.reshape(n, h), edge_index, zeros_hbm)
    outp = _final_pairs(qp, s2.reshape(2, n2, h2), b2pair)
    return outp.reshape(n, c_out)


# speedup vs baseline: 15.6589x; 1.0413x over previous
import functools

import jax
import jax.numpy as jnp
from jax import lax
from jax.experimental import pallas as pl
from jax.experimental.pallas import tpu as pltpu
from jax.experimental.pallas import tpu_sc as plsc


def _matmul(x, w):
    n, d = x.shape
    h = w.shape[1]
    bn = 2000
    def body(x_ref, w_ref, o_ref):
        o_ref[...] = jnp.dot(x_ref[...], w_ref[...],
                             preferred_element_type=jnp.float32)
    return pl.pallas_call(
        body,
        grid=(n // bn,),
        in_specs=[pl.BlockSpec((bn, d), lambda i: (i, 0)),
                  pl.BlockSpec((d, h), lambda i: (0, 0))],
        out_specs=pl.BlockSpec((bn, h), lambda i: (i, 0)),
        out_shape=jax.ShapeDtypeStruct((n, h), jnp.float32),
    )(x, w)


def _mid_pairs(pp, s1p, b1pair, w2d):
    n2, h2 = pp.shape
    bn = 1000
    def body(p_ref, s_ref, b_ref, w_ref, q_ref):
        hh = p_ref[...] + s_ref[0] + s_ref[1] + b_ref[...]
        hh = jnp.maximum(hh, 0.0)
        q_ref[...] = jnp.dot(hh, w_ref[...],
                             preferred_element_type=jnp.float32)
    return pl.pallas_call(
        body,
        grid=(n2 // bn,),
        in_specs=[pl.BlockSpec((bn, h2), lambda i: (i, 0)),
                  pl.BlockSpec((2, bn, h2), lambda i: (0, i, 0)),
                  pl.BlockSpec((1, h2), lambda i: (0, 0)),
                  pl.BlockSpec((h2, h2), lambda i: (0, 0))],
        out_specs=pl.BlockSpec((bn, h2), lambda i: (i, 0)),
        out_shape=jax.ShapeDtypeStruct((n2, h2), jnp.float32),
    )(pp, s1p, b1pair, w2d)


def _final_pairs(qp, s2p, b2pair):
    n2, h2 = qp.shape
    bn = 1000
    def body(q_ref, s_ref, b_ref, o_ref):
        o_ref[...] = q_ref[...] + s_ref[0] + s_ref[1] + b_ref[...]
    return pl.pallas_call(
        body,
        grid=(n2 // bn,),
        in_specs=[pl.BlockSpec((bn, h2), lambda i: (i, 0)),
                  pl.BlockSpec((2, bn, h2), lambda i: (0, i, 0)),
                  pl.BlockSpec((1, h2), lambda i: (0, 0))],
        out_specs=pl.BlockSpec((bn, h2), lambda i: (i, 0)),
        out_shape=jax.ShapeDtypeStruct((n2, h2), jnp.float32),
    )(qp, s2p, b2pair)


def _sc_segment_sum(table, ei, zeros_hbm):
    n, h = table.shape
    e = ei.shape[1]
    info = plsc.get_sparse_core_info()
    nc, ns = info.num_cores, info.num_subcores
    nw = nc * ns
    ch = 128
    nchunk = e // ch
    kmax = -(-nchunk // nw)
    blk = 200
    nblk = n // blk
    kblk = -(-nblk // ns)
    nb = 7

    mesh = plsc.VectorSubcoreMesh(core_axis_name="c", subcore_axis_name="s")

    @functools.partial(
        pl.kernel,
        out_type=jax.ShapeDtypeStruct((nc, n, h), jnp.float32),
        mesh=mesh,
        compiler_params=pltpu.CompilerParams(use_tc_tiling_on_sc=False),
        scratch_types=[
            pltpu.VMEM((nb, 2, ch), jnp.int32),
            pltpu.VMEM((nb, ch, h), jnp.float32),
            pltpu.VMEM_SHARED((n, h), jnp.float32),
            pltpu.SemaphoreType.DMA((nb,)),
            pltpu.SemaphoreType.DMA((nb,)),
            pltpu.SemaphoreType.DMA((nb,)),
        ],
    )
    def k(table_hbm, ei_hbm, zero_hbm, out_hbm,
          ii, rows, acc, sem_i, sem_g, sem_s):
        c = lax.axis_index("c")
        s = lax.axis_index("s")
        w = c * ns + s

        def zbody(jj, carry):
            b = s + jj * ns
            @pl.when(b < nblk)
            def _():
                r0 = b * blk
                pltpu.sync_copy(zero_hbm.at[pl.ds(r0, blk)],
                                acc.at[pl.ds(r0, blk)])
            return carry

        lax.fori_loop(0, kblk, zbody, 0)
        plsc.subcore_barrier()

        def fire_idx(kk):
            m = lax.rem(kk, nb)
            e0 = (w + kk * nw) * ch
            pltpu.async_copy(ei_hbm.at[:, pl.ds(e0, ch)], ii.at[m],
                             sem_i.at[m])

        def fire_gather(kk):
            m = lax.rem(kk, nb)
            pltpu.make_async_copy(ei_hbm.at[:, pl.ds(0, ch)], ii.at[m],
                                  sem_i.at[m]).wait()
            pltpu.async_copy(table_hbm.at[ii.at[m, 0]], rows.at[m],
                             sem_g.at[m])

        def valid(kk):
            return (kk >= 0) & (w + kk * nw < nchunk)

        for kk0 in range(5):
            @pl.when(valid(kk0))
            def _():
                fire_idx(kk0)
        for kk0 in range(3):
            @pl.when(valid(kk0))
            def _():
                fire_gather(kk0)

        def body(kk, carry):
            m = lax.rem(kk, nb)
            m5 = lax.rem(kk + 5, nb)
            @pl.when(valid(kk - 2))
            def _():
                pltpu.make_async_copy(rows.at[m5], acc.at[ii.at[m5, 1]],
                                      sem_s.at[m5]).wait()
            @pl.when(valid(kk + 5))
            def _():
                fire_idx(kk + 5)
            @pl.when(valid(kk + 3))
            def _():
                fire_gather(kk + 3)
            @pl.when(valid(kk))
            def _():
                pltpu.make_async_copy(table_hbm.at[ii.at[m, 0]], rows.at[m],
                                      sem_g.at[m]).wait()
                pltpu.async_copy(rows.at[m], acc.at[ii.at[m, 1]], sem_s.at[m],
                                 add=True)
            return carry

        lax.fori_loop(0, kmax + 2, body, 0)
        plsc.subcore_barrier()

        def obody(jj, carry):
            b = s + jj * ns
            @pl.when(b < nblk)
            def _():
                r0 = b * blk
                pltpu.sync_copy(acc.at[pl.ds(r0, blk)],
                                out_hbm.at[c, pl.ds(r0, blk)])
            return carry

        lax.fori_loop(0, kblk, obody, 0)

    return k(table, ei, zeros_hbm)


def kernel(x, edge_index, W1, b1, W2, b2):
    n = x.shape[0]
    h = W1.shape[1]
    c_out = W2.shape[1]
    n2 = n // 2
    h2 = 2 * h

    w2d = jnp.zeros((h2, h2), W2.dtype)
    w2d = w2d.at[:h, :c_out].set(W2).at[h:, c_out:].set(W2)
    b1pair = jnp.concatenate([b1, b1]).reshape(1, h2)
    b2pair = jnp.concatenate([b2, b2]).reshape(1, h2)
    zeros_hbm = jnp.zeros((n, h), jnp.float32)

    p = _matmul(x, W1)
    pp = p.reshape(n2, h2)
    s1 = _sc_segment_sum(p, edge_index, zeros_hbm)
    qp = _mid_pairs(pp, s1.reshape(2, n2, h2), b1pair, w2d)
    s2 = _sc_segment_sum(qp.reshape(n, h), edge_index, zeros_hbm)
    outp = _final_pairs(qp, s2.reshape(2, n2, h2), b2pair)
    return outp.reshape(n, c_out)
